# Initial kernel scaffold; baseline (speedup 1.0000x reference)
import jax
import jax.numpy as jnp
from jax.experimental import pallas as pl


def kernel(eb_nodes, eb_edges, numb_nodes, numb_edges, theta_obj, theta_pred):
    raise NotImplementedError("write your pallas kernel here")


# speedup vs baseline: 13.0203x; 13.0203x over previous
import functools

import jax
import jax.numpy as jnp
from jax.experimental import pallas as pl
from jax.experimental.pallas import tpu as pltpu

NODE_DIM = 512
BATCH = 16
TOTAL = 32768
BLK = 1024
NBLK = TOTAL // BLK


def _onehot(pos, s_ref, c_ref):
    s = s_ref[...]
    c = c_ref[...]
    return ((pos >= s) & (pos < s + c)).astype(jnp.float32)


def _pass1_body(xn_ref, xe_ref, sn_ref, cn_ref, se_ref, ce_ref,
                outn_ref, oute_ref, accn, acce):
    i = pl.program_id(0)
    pos = jax.lax.broadcasted_iota(jnp.int32, (BATCH, BLK), 1) + i * BLK
    mn = _onehot(pos, sn_ref, cn_ref)
    me = _onehot(pos, se_ref, ce_ref)
    dn = (((1,), (0,)), ((), ()))
    pn = jax.lax.dot_general(mn, xn_ref[...], dn, preferred_element_type=jnp.float32)
    pe = jax.lax.dot_general(me, xe_ref[...], dn, preferred_element_type=jnp.float32)

    @pl.when(i == 0)
    def _():
        accn[...] = pn
        acce[...] = pe

    @pl.when(i > 0)
    def _():
        accn[...] += pn
        acce[...] += pe

    @pl.when(i == pl.num_programs(0) - 1)
    def _():
        outn_ref[...] = accn[...]
        oute_ref[...] = acce[...]


def _pass2_body(sumn_ref, sume_ref, to_ref, tp_ref,
                sn_ref, cn_ref, se_ref, ce_ref,
                xn_ref, xe_ref, out_ref, accn, acce, comn, come):
    i = pl.program_id(0)
    dnums_t = (((1,), (1,)), ((), ()))
    dnums_m = (((1,), (0,)), ((), ()))

    @pl.when(i == 0)
    def _():
        den_n = jnp.maximum(cn_ref[...].astype(jnp.float32), 1.0)
        den_e = jnp.maximum(ce_ref[...].astype(jnp.float32), 1.0)
        mean_n = sumn_ref[...] / den_n
        mean_e = sume_ref[...] / den_e
        comn[...] = jnp.maximum(
            jax.lax.dot_general(mean_n, to_ref[...], dnums_t,
                                preferred_element_type=jnp.float32), 0.0)
        come[...] = jnp.maximum(
            jax.lax.dot_general(mean_e, tp_ref[...], dnums_t,
                                preferred_element_type=jnp.float32), 0.0)
        accn[...] = jnp.zeros_like(accn)
        acce[...] = jnp.zeros_like(acce)

    @pl.when(i > 0)
    def _():
        pos = jax.lax.broadcasted_iota(jnp.int32, (BATCH, BLK), 1) + (i - 1) * BLK
        mn = _onehot(pos, sn_ref, cn_ref)
        me = _onehot(pos, se_ref, ce_ref)
        xn = xn_ref[...]
        xe = xe_ref[...]
        s16n = jax.lax.dot_general(comn[...], xn, dnums_t,
                                   preferred_element_type=jnp.float32)
        s16e = jax.lax.dot_general(come[...], xe, dnums_t,
                                   preferred_element_type=jnp.float32)
        gn = mn / (1.0 + jnp.exp(-s16n))
        ge = me / (1.0 + jnp.exp(-s16e))
        accn[...] += jax.lax.dot_general(gn, xn, dnums_m,
                                         preferred_element_type=jnp.float32)
        acce[...] += jax.lax.dot_general(ge, xe, dnums_m,
                                         preferred_element_type=jnp.float32)

    @pl.when(i == pl.num_programs(0) - 1)
    def _():
        den_n = jnp.maximum(cn_ref[...].astype(jnp.float32), 1.0)
        den_e = jnp.maximum(ce_ref[...].astype(jnp.float32), 1.0)
        out_ref[:, :NODE_DIM] = accn[...] / den_n
        out_ref[:, NODE_DIM:] = acce[...] / den_e


def kernel(eb_nodes, eb_edges, numb_nodes, numb_edges, theta_obj, theta_pred):
    starts_n = (jnp.cumsum(numb_nodes) - numb_nodes).astype(jnp.int32).reshape(BATCH, 1)
    starts_e = (jnp.cumsum(numb_edges) - numb_edges).astype(jnp.int32).reshape(BATCH, 1)
    counts_n = numb_nodes.reshape(BATCH, 1)
    counts_e = numb_edges.reshape(BATCH, 1)

    small = pl.BlockSpec((BATCH, 1), lambda i: (0, 0))
    xspec1 = pl.BlockSpec((BLK, NODE_DIM), lambda i: (i, 0))
    full = lambda shp: pl.BlockSpec(shp, lambda i: (0,) * len(shp))

    sums_n, sums_e = pl.pallas_call(
        _pass1_body,
        grid=(NBLK,),
        in_specs=[xspec1, xspec1, small, small, small, small],
        out_specs=[full((BATCH, NODE_DIM)), full((BATCH, NODE_DIM))],
        out_shape=[jax.ShapeDtypeStruct((BATCH, NODE_DIM), jnp.float32)] * 2,
        scratch_shapes=[pltpu.VMEM((BATCH, NODE_DIM), jnp.float32)] * 2,
        compiler_params=pltpu.CompilerParams(
            dimension_semantics=("arbitrary",)),
    )(eb_nodes, eb_edges, starts_n, counts_n, starts_e, counts_e)

    xspec2 = pl.BlockSpec((BLK, NODE_DIM), lambda i: (jnp.maximum(i - 1, 0), 0))
    geb = pl.pallas_call(
        _pass2_body,
        grid=(NBLK + 1,),
        in_specs=[full((BATCH, NODE_DIM)), full((BATCH, NODE_DIM)),
                  full((NODE_DIM, NODE_DIM)), full((NODE_DIM, NODE_DIM)),
                  small, small, small, small,
                  xspec2, xspec2],
        out_specs=full((BATCH, 2 * NODE_DIM)),
        out_shape=jax.ShapeDtypeStruct((BATCH, 2 * NODE_DIM), jnp.float32),
        scratch_shapes=[pltpu.VMEM((BATCH, NODE_DIM), jnp.float32)] * 4,
        compiler_params=pltpu.CompilerParams(
            dimension_semantics=("arbitrary",)),
    )(sums_n, sums_e, theta_obj, theta_pred,
      starts_n, counts_n, starts_e, counts_e, eb_nodes, eb_edges)
    return geb
